# baseline (speedup 1.0000x reference)
import functools

import jax
import jax.numpy as jnp
from jax import lax
from jax.experimental import pallas as pl
from jax.experimental.pallas import tpu as pltpu
from jax.experimental.pallas import tpu_sc as plsc


@functools.lru_cache(maxsize=None)
def _build_sc_gather(bp, ih, iw, W, L, H):
    P = bp * ih * iw
    ppb = ih * iw
    info = plsc.get_sparse_core_info()
    NC, NS = info.num_cores, info.num_subcores
    NW = NC * NS
    ppw = P // NW
    C = 8192
    nchunks = ppw // C
    wpb = NW // bp
    LH = L * H

    mesh = plsc.VectorSubcoreMesh(core_axis_name="c", subcore_axis_name="s")

    @functools.partial(
        pl.kernel,
        out_type=jax.ShapeDtypeStruct((P,), jnp.float32),
        mesh=mesh,
        scratch_types=[
            pltpu.VMEM((C,), jnp.float32),
            pltpu.VMEM((C,), jnp.float32),
            pltpu.VMEM((C,), jnp.float32),
            pltpu.VMEM((C,), jnp.int32),
            pltpu.VMEM((C,), jnp.float32),
            pltpu.VMEM((4, 16), jnp.float32),
            pltpu.VMEM((6, 16), jnp.int32),
            pltpu.SemaphoreType.DMA,
        ],
    )
    def sc_kernel(pts_hbm, tbl_hbm, fpar_hbm, ipar_hbm, out_hbm,
                  xv, yv, zv, idxv, resv, fpv, ipv, sem):
        wid = lax.axis_index("s") * NC + lax.axis_index("c")
        b = wid // wpb
        woff = (wid % wpb) * ppw
        base = b * 3 * ppb

        pltpu.sync_copy(fpar_hbm, fpv)
        pltpu.sync_copy(ipar_hbm, ipv)
        ox = fpv[0, :]
        oy = fpv[1, :]
        oz = fpv[2, :]
        vs = fpv[3, :]
        mnx = ipv[0, :]
        mny = ipv[1, :]
        mnz = ipv[2, :]
        mxx = ipv[3, :]
        mxy = ipv[4, :]
        mxz = ipv[5, :]

        def chunk(k, carry):
            src = woff + k * C
            pltpu.sync_copy(pts_hbm.at[pl.ds(base + src, C)], xv)
            pltpu.sync_copy(pts_hbm.at[pl.ds(base + ppb + src, C)], yv)
            pltpu.sync_copy(pts_hbm.at[pl.ds(base + 2 * ppb + src, C)], zv)

            def vec(i, c2):
                s = pl.ds(i * 16, 16)
                ix = ((xv[s] - ox) / vs + 0.5).astype(jnp.int32)
                iy = ((yv[s] - oy) / vs + 0.5).astype(jnp.int32)
                iz = ((zv[s] - oz) / vs + 0.5).astype(jnp.int32)
                m = ((ix >= mnx) & (ix < mxx)
                     & (iy >= mny) & (iy < mxy)
                     & (iz >= mnz) & (iz < mxz))
                flat = ix * LH + iy * H + iz
                idxv[s] = jnp.where(m, flat, 0)
                return c2

            lax.fori_loop(0, C // 16, vec, 0)
            pltpu.async_copy(tbl_hbm.at[idxv], resv, sem).wait()
            pltpu.sync_copy(resv, out_hbm.at[pl.ds(b * ppb + src, C)])
            return carry

        lax.fori_loop(0, nchunks, chunk, 0)

    return sc_kernel


def kernel(point_coordinates, voxelgrid_data, origin, voxel_size,
           min_bounds, max_bounds):
    bp, _, ih, iw = point_coordinates.shape
    _, _, W, L, H = voxelgrid_data.shape

    pts_flat = point_coordinates.reshape(-1)
    tbl_flat = voxelgrid_data.reshape(-1)
    fpar = jnp.stack([
        jnp.broadcast_to(origin[0, 0], (16,)),
        jnp.broadcast_to(origin[0, 1], (16,)),
        jnp.broadcast_to(origin[0, 2], (16,)),
        jnp.broadcast_to(voxel_size[0], (16,)),
    ]).astype(jnp.float32)
    ipar = jnp.stack([
        jnp.broadcast_to(min_bounds[0], (16,)),
        jnp.broadcast_to(min_bounds[1], (16,)),
        jnp.broadcast_to(min_bounds[2], (16,)),
        jnp.broadcast_to(max_bounds[0], (16,)),
        jnp.broadcast_to(max_bounds[1], (16,)),
        jnp.broadcast_to(max_bounds[2], (16,)),
    ]).astype(jnp.int32)

    sc = _build_sc_gather(bp, ih, iw, W, L, H)
    out = sc(pts_flat, tbl_flat, fpar, ipar)
    return out.reshape(bp, 1, ih, iw)
